# Initial kernel scaffold; baseline (speedup 1.0000x reference)
import jax
import jax.numpy as jnp
from jax.experimental import pallas as pl


def kernel(data_idx, adj_matrix, edge_rel, embeddings, neighbor_idx, done_w, W1, b1, Wa, ba, g, beta, Wb, bb):
    raise NotImplementedError("write your pallas kernel here")


# speedup vs baseline: 1.7355x; 1.7355x over previous
import jax
import jax.numpy as jnp
from jax import lax
from jax.experimental import pallas as pl
from jax.experimental.pallas import tpu as pltpu
from jax.experimental.pallas import tpu_sc as plsc

N = 10000
D = 128
B = 1024
M = 64
H = 32
NPAD = 10008

NC, NS = 2, 16
NW = NC * NS
BM = B * M
BP = B * 2
NB_W = BM // NW
NCHUNK = 2
NB_C = NB_W // NCHUNK
PB_W = BP // NW

QB = 64
RB = QB * M


def _sc_gather(gef_hbm, geb_hbm, nidx_hbm, didx_hbm, out_n, out_p,
               idx_v, rows_v, idx2_v, rows2_v, sem, sem2):
    wid = lax.axis_index("s") * NC + lax.axis_index("c")
    base2 = wid * PB_W
    pltpu.sync_copy(didx_hbm.at[pl.ds(base2, PB_W)], idx2_v)
    cp2 = pltpu.async_copy(gef_hbm.at[idx2_v], rows2_v, sem2)
    for c in range(NCHUNK):
        base = wid * NB_W + c * NB_C
        pltpu.sync_copy(nidx_hbm.at[pl.ds(base, NB_C)], idx_v)
        pltpu.async_copy(geb_hbm.at[idx_v], rows_v, sem).wait()
        pltpu.sync_copy(rows_v, out_n.at[pl.ds(base, NB_C)])
    cp2.wait()
    pltpu.sync_copy(rows2_v, out_p.at[pl.ds(base2, PB_W)])


def _mlp_body(pair_ref, nbf_ref, w1_ref, b1_ref, wa_ref, ba_ref, g_ref,
              beta_ref, wb_ref, lg_ref):
    pr = pair_ref[...].reshape(QB, 2, D)
    kq = (pr[:, 0, :] + pr[:, 1, :]) * 0.5
    e1b = kq.astype(jnp.bfloat16)
    e1r = jnp.broadcast_to(e1b[:, None, :], (QB, M, D)).reshape(RB, D)
    kqr = jnp.broadcast_to(kq[:, None, :], (QB, M, D)).reshape(RB, D)
    cat = jnp.concatenate([e1r, nbf_ref[...]], axis=1)
    c1 = jnp.dot(cat, w1_ref[...], preferred_element_type=jnp.float32)
    e = kqr + (c1 + b1_ref[...])
    ebf = e.astype(jnp.bfloat16)
    c2 = jnp.dot(ebf, wa_ref[...], preferred_element_type=jnp.float32)
    h = c2 + ba_ref[...]
    mu = jnp.mean(h, axis=-1, keepdims=True)
    var = jnp.mean((h - mu) ** 2, axis=-1, keepdims=True)
    hn = (h - mu) / jnp.sqrt(var + 1e-5) * g_ref[...] + beta_ref[...]
    t = jnp.tanh(hn)
    lg_ref[...] = jnp.dot(t, wb_ref[...], preferred_element_type=jnp.float32)


def _sample_body(lg_ref, bb_ref, gum_ref, nidx_ref, probs_ref, samp_ref):
    l = lg_ref[...] + bb_ref[0, 0]
    mx = jnp.max(l, axis=-1, keepdims=True)
    ex = jnp.exp(l - mx)
    s = jnp.sum(ex, axis=-1, keepdims=True)
    probs = ex / s
    probs_ref[...] = probs
    z = gum_ref[...] + jnp.log(probs + 1e-20)
    zmax = jnp.max(z, axis=-1, keepdims=True)
    iota = lax.broadcasted_iota(jnp.int32, (B, M), 1)
    samp = jnp.min(jnp.where(z == zmax, iota, M), axis=-1)
    samp_ref[...] = jnp.sum(
        jnp.where(iota == samp[:, None], nidx_ref[...], 0),
        axis=-1, keepdims=True)


def kernel(data_idx, adj_matrix, edge_rel, embeddings, neighbor_idx, done_w,
           W1, b1, Wa, ba, g, beta, Wb, bb):
    del adj_matrix, edge_rel
    f32 = jnp.float32

    gef = jnp.concatenate(
        [embeddings, done_w, jnp.zeros((NPAD - N - 1, D), f32)], axis=0)
    geb = gef.astype(jnp.bfloat16)

    nidx_flat = neighbor_idx.reshape(BM).astype(jnp.int32)
    didx_flat = data_idx.reshape(BP).astype(jnp.int32)

    mesh = plsc.VectorSubcoreMesh(core_axis_name="c", subcore_axis_name="s")
    gathered_n, gathered_p = pl.kernel(
        _sc_gather,
        mesh=mesh,
        compiler_params=pltpu.CompilerParams(use_tc_tiling_on_sc=False),
        out_type=[jax.ShapeDtypeStruct((BM, D), jnp.bfloat16),
                  jax.ShapeDtypeStruct((BP, D), f32)],
        scratch_types=[
            pltpu.VMEM((NB_C,), jnp.int32),
            pltpu.VMEM((NB_C, D), jnp.bfloat16),
            pltpu.VMEM((PB_W,), jnp.int32),
            pltpu.VMEM((PB_W, D), f32),
            pltpu.SemaphoreType.DMA,
            pltpu.SemaphoreType.DMA,
        ],
    )(gef, geb, nidx_flat, didx_flat)

    logits_v = pl.pallas_call(
        _mlp_body,
        grid=(B // QB,),
        in_specs=[
            pl.BlockSpec((2 * QB, D), lambda i: (i, 0)),
            pl.BlockSpec((RB, D), lambda i: (i, 0)),
            pl.BlockSpec((2 * D, D), lambda i: (0, 0)),
            pl.BlockSpec((1, D), lambda i: (0, 0)),
            pl.BlockSpec((D, H), lambda i: (0, 0)),
            pl.BlockSpec((1, H), lambda i: (0, 0)),
            pl.BlockSpec((1, H), lambda i: (0, 0)),
            pl.BlockSpec((1, H), lambda i: (0, 0)),
            pl.BlockSpec((H, 1), lambda i: (0, 0)),
        ],
        out_specs=pl.BlockSpec((RB, 1), lambda i: (i, 0)),
        out_shape=jax.ShapeDtypeStruct((BM, 1), f32),
    )(gathered_p, gathered_n, W1, b1.reshape(1, D), Wa, ba.reshape(1, H),
      g.reshape(1, H), beta.reshape(1, H), Wb)

    gum = jax.random.gumbel(jax.random.key(42), (B, M), f32)

    probs, sampled = pl.pallas_call(
        _sample_body,
        out_shape=[jax.ShapeDtypeStruct((B, M), f32),
                   jax.ShapeDtypeStruct((B, 1), jnp.int32)],
    )(logits_v.reshape(B, M), bb.reshape(1, 1), gum,
      neighbor_idx.astype(jnp.int32))

    return (probs, sampled.reshape(B))


# speedup vs baseline: 2.7850x; 1.6047x over previous
import jax
import jax.numpy as jnp
from jax import lax
from jax.experimental import pallas as pl
from jax.experimental.pallas import tpu as pltpu
from jax.experimental.pallas import tpu_sc as plsc

N = 10000
D = 128
B = 1024
M = 64
H = 32
NPAD = 10008

NC, NS = 2, 16
NW = NC * NS
BM = B * M
BP = B * 2
NB_W = BM // NW
NCHUNK = 8
CROWS = NB_W // NCHUNK
PB_W = BP // NW

QB = 64
RB = QB * M


def _sc_gather(gef_hbm, nidx_hbm, didx_hbm, out_n, out_p,
               idxn_v0, idxn_v1, idxp_v, nbuf0, nbuf1, pair_v,
               semg0, semg1, semp):
    wid = lax.axis_index("s") * NC + lax.axis_index("c")
    pltpu.sync_copy(didx_hbm.at[pl.ds(wid * PB_W, PB_W)], idxp_v)
    cpp = pltpu.async_copy(gef_hbm.at[idxp_v], pair_v, semp)
    nbufs = (nbuf0, nbuf1)
    idxvs = (idxn_v0, idxn_v1)
    semgs = (semg0, semg1)
    for c in range(NCHUNK):
        cur = c % 2
        base = wid * NB_W + c * CROWS
        pltpu.sync_copy(nidx_hbm.at[pl.ds(base, CROWS)], idxvs[cur])
        pltpu.async_copy(
            gef_hbm.at[idxvs[cur]], nbufs[cur], semgs[cur]).wait()
        pltpu.sync_copy(nbufs[cur], out_n.at[pl.ds(base, CROWS)])
    cpp.wait()
    pltpu.sync_copy(pair_v, out_p.at[pl.ds(wid * PB_W, PB_W)])


def _mlp_body(pair_ref, ngf_ref, w1_ref, b1_ref, wa_ref, ba_ref, g_ref,
              beta_ref, wb_ref, lg_ref):
    pr = pair_ref[...].reshape(QB, 2, D)
    kq = (pr[:, 0, :] + pr[:, 1, :]) * 0.5
    e1b = kq.astype(jnp.bfloat16)
    e1r = jnp.broadcast_to(e1b[:, None, :], (QB, M, D)).reshape(RB, D)
    kqr = jnp.broadcast_to(kq[:, None, :], (QB, M, D)).reshape(RB, D)
    nbf = ngf_ref[...].astype(jnp.bfloat16)
    cat = jnp.concatenate([e1r, nbf], axis=1)
    c1 = jnp.dot(cat, w1_ref[...], preferred_element_type=jnp.float32)
    e = kqr + (c1 + b1_ref[...])
    ebf = e.astype(jnp.bfloat16)
    c2 = jnp.dot(ebf, wa_ref[...], preferred_element_type=jnp.float32)
    h = c2 + ba_ref[...]
    mu = jnp.mean(h, axis=-1, keepdims=True)
    var = jnp.mean((h - mu) ** 2, axis=-1, keepdims=True)
    hn = (h - mu) / jnp.sqrt(var + 1e-5) * g_ref[...] + beta_ref[...]
    t = jnp.tanh(hn)
    c3 = jnp.dot(t, wb_ref[...], preferred_element_type=jnp.float32)
    lg_ref[...] = c3.reshape(QB, M)


def _sample_body(lg_ref, bb_ref, gum_ref, nidx_ref, probs_ref, samp_ref):
    l = lg_ref[...] + bb_ref[0, 0]
    mx = jnp.max(l, axis=-1, keepdims=True)
    ex = jnp.exp(l - mx)
    s = jnp.sum(ex, axis=-1, keepdims=True)
    probs = ex / s
    probs_ref[...] = probs
    z = gum_ref[...] + jnp.log(probs + 1e-20)
    zmax = jnp.max(z, axis=-1, keepdims=True)
    iota = lax.broadcasted_iota(jnp.int32, (B, M), 1)
    samp = jnp.min(jnp.where(z == zmax, iota, M), axis=-1)
    samp_ref[...] = jnp.sum(
        jnp.where(iota == samp[:, None], nidx_ref[...], 0),
        axis=-1, keepdims=True)


def kernel(data_idx, adj_matrix, edge_rel, embeddings, neighbor_idx, done_w,
           W1, b1, Wa, ba, g, beta, Wb, bb):
    del adj_matrix, edge_rel
    f32 = jnp.float32

    gef = jnp.concatenate(
        [embeddings, done_w, jnp.zeros((NPAD - N - 1, D), f32)], axis=0)

    nidx_flat = neighbor_idx.reshape(BM).astype(jnp.int32)
    didx_flat = data_idx.reshape(BP).astype(jnp.int32)

    mesh = plsc.VectorSubcoreMesh(core_axis_name="c", subcore_axis_name="s")
    gathered_n, gathered_p = pl.kernel(
        _sc_gather,
        mesh=mesh,
        compiler_params=pltpu.CompilerParams(use_tc_tiling_on_sc=False),
        out_type=[jax.ShapeDtypeStruct((BM, D), f32),
                  jax.ShapeDtypeStruct((BP, D), f32)],
        scratch_types=[
            pltpu.VMEM((CROWS,), jnp.int32),
            pltpu.VMEM((CROWS,), jnp.int32),
            pltpu.VMEM((PB_W,), jnp.int32),
            pltpu.VMEM((CROWS, D), f32),
            pltpu.VMEM((CROWS, D), f32),
            pltpu.VMEM((PB_W, D), f32),
            pltpu.SemaphoreType.DMA,
            pltpu.SemaphoreType.DMA,
            pltpu.SemaphoreType.DMA,
        ],
    )(gef, nidx_flat, didx_flat)

    logits = pl.pallas_call(
        _mlp_body,
        grid=(B // QB,),
        in_specs=[
            pl.BlockSpec((2 * QB, D), lambda i: (i, 0)),
            pl.BlockSpec((RB, D), lambda i: (i, 0)),
            pl.BlockSpec((2 * D, D), lambda i: (0, 0)),
            pl.BlockSpec((1, D), lambda i: (0, 0)),
            pl.BlockSpec((D, H), lambda i: (0, 0)),
            pl.BlockSpec((1, H), lambda i: (0, 0)),
            pl.BlockSpec((1, H), lambda i: (0, 0)),
            pl.BlockSpec((1, H), lambda i: (0, 0)),
            pl.BlockSpec((H, 1), lambda i: (0, 0)),
        ],
        out_specs=pl.BlockSpec((QB, M), lambda i: (i, 0)),
        out_shape=jax.ShapeDtypeStruct((B, M), f32),
    )(gathered_p, gathered_n, W1, b1.reshape(1, D), Wa, ba.reshape(1, H),
      g.reshape(1, H), beta.reshape(1, H), Wb)

    gum = jax.random.gumbel(jax.random.key(42), (B, M), f32)

    probs, sampled = pl.pallas_call(
        _sample_body,
        out_shape=[jax.ShapeDtypeStruct((B, M), f32),
                   jax.ShapeDtypeStruct((B, 1), jnp.int32)],
    )(logits, bb.reshape(1, 1), gum, neighbor_idx.astype(jnp.int32))

    return (probs, sampled.reshape(B))


# speedup vs baseline: 3.2076x; 1.1518x over previous
import jax
import jax.numpy as jnp
from jax import lax
from jax.experimental import pallas as pl
from jax.experimental.pallas import tpu as pltpu
from jax.experimental.pallas import tpu_sc as plsc

N = 10000
D = 128
B = 1024
M = 64
H = 32
NPAD = 10008

NC, NS = 2, 16
NW = NC * NS
BM = B * M
BP = B * 2
NB_W = BM // NW
NCHUNK = 8
CROWS = NB_W // NCHUNK
PB_W = BP // NW

QB = 64
RB = QB * M


def _sc_gather(gef_hbm, nidx_hbm, didx_hbm, out_n, out_p,
               idxn_v0, idxn_v1, idxp_v, nbuf0, nbuf1, pair_v,
               semg0, semg1, semw0, semw1, semp):
    wid = lax.axis_index("s") * NC + lax.axis_index("c")
    pltpu.sync_copy(didx_hbm.at[pl.ds(wid * PB_W, PB_W)], idxp_v)
    cpp = pltpu.async_copy(gef_hbm.at[idxp_v], pair_v, semp)
    nbufs = (nbuf0, nbuf1)
    idxvs = (idxn_v0, idxn_v1)
    semgs = (semg0, semg1)
    semws = (semw0, semw1)
    cpg = [None, None]
    cpw = [None, None]
    pltpu.sync_copy(nidx_hbm.at[pl.ds(wid * NB_W, CROWS)], idxvs[0])
    cpg[0] = pltpu.async_copy(gef_hbm.at[idxvs[0]], nbufs[0], semgs[0])
    for c in range(NCHUNK):
        cur = c % 2
        nxt = (c + 1) % 2
        if c + 1 < NCHUNK:
            pltpu.sync_copy(
                nidx_hbm.at[pl.ds(wid * NB_W + (c + 1) * CROWS, CROWS)],
                idxvs[nxt])
            if cpw[nxt] is not None:
                cpw[nxt].wait()
            cpg[nxt] = pltpu.async_copy(
                gef_hbm.at[idxvs[nxt]], nbufs[nxt], semgs[nxt])
        cpg[cur].wait()
        cpw[cur] = pltpu.async_copy(
            nbufs[cur], out_n.at[pl.ds(wid * NB_W + c * CROWS, CROWS)],
            semws[cur])
    cpw[0].wait()
    cpw[1].wait()
    cpp.wait()
    pltpu.sync_copy(pair_v, out_p.at[pl.ds(wid * PB_W, PB_W)])


def _mlp_body(pair_ref, ngf_ref, w1_ref, b1_ref, wa_ref,
              ba_ref, g_ref, beta_ref, wb_ref, lg_ref):
    pr = pair_ref[...].reshape(QB, 2, D)
    kq = (pr[:, 0, :] + pr[:, 1, :]) * 0.5
    e1b = kq.astype(jnp.bfloat16)
    e1r = jnp.broadcast_to(e1b[:, None, :], (QB, M, D)).reshape(RB, D)
    kqr = jnp.broadcast_to(kq[:, None, :], (QB, M, D)).reshape(RB, D)
    nbf = ngf_ref[...].astype(jnp.bfloat16)
    cat = jnp.concatenate([e1r, nbf], axis=1)
    c1 = jnp.dot(cat, w1_ref[...], preferred_element_type=jnp.float32)
    e = kqr + (c1 + b1_ref[...])
    ebf = e.astype(jnp.bfloat16)
    c2 = jnp.dot(ebf, wa_ref[...], preferred_element_type=jnp.float32)
    h = c2 + ba_ref[...]
    mu = jnp.mean(h, axis=-1, keepdims=True)
    var = jnp.mean((h - mu) ** 2, axis=-1, keepdims=True)
    hn = (h - mu) / jnp.sqrt(var + 1e-5) * g_ref[...] + beta_ref[...]
    t = jnp.tanh(hn)
    c3 = jnp.dot(t, wb_ref[...], preferred_element_type=jnp.float32)
    lg_ref[...] = c3.reshape(QB, M)


def _sample_body(lg_ref, bb_ref, gum_ref, nidx_ref, probs_ref, samp_ref):
    l = lg_ref[...] + bb_ref[0, 0]
    mx = jnp.max(l, axis=-1, keepdims=True)
    ex = jnp.exp(l - mx)
    s = jnp.sum(ex, axis=-1, keepdims=True)
    probs = ex / s
    probs_ref[...] = probs
    z = gum_ref[...] + jnp.log(probs + 1e-20)
    zmax = jnp.max(z, axis=-1, keepdims=True)
    iota = lax.broadcasted_iota(jnp.int32, (B, M), 1)
    samp = jnp.min(jnp.where(z == zmax, iota, M), axis=-1)
    samp_ref[...] = jnp.sum(
        jnp.where(iota == samp[:, None], nidx_ref[...], 0),
        axis=-1, keepdims=True)


def kernel(data_idx, adj_matrix, edge_rel, embeddings, neighbor_idx, done_w,
           W1, b1, Wa, ba, g, beta, Wb, bb):
    del adj_matrix, edge_rel
    f32 = jnp.float32

    gef = jnp.concatenate([embeddings, done_w], axis=0)
    nidx_flat = neighbor_idx.reshape(BM).astype(jnp.int32)
    didx_flat = data_idx.reshape(BP).astype(jnp.int32)

    mesh = plsc.VectorSubcoreMesh(core_axis_name="c", subcore_axis_name="s")
    gathered_n, gathered_p = pl.kernel(
        _sc_gather,
        mesh=mesh,
        compiler_params=pltpu.CompilerParams(use_tc_tiling_on_sc=False),
        out_type=[jax.ShapeDtypeStruct((BM, D), f32),
                  jax.ShapeDtypeStruct((BP, D), f32)],
        scratch_types=[
            pltpu.VMEM((CROWS,), jnp.int32),
            pltpu.VMEM((CROWS,), jnp.int32),
            pltpu.VMEM((PB_W,), jnp.int32),
            pltpu.VMEM((CROWS, D), f32),
            pltpu.VMEM((CROWS, D), f32),
            pltpu.VMEM((PB_W, D), f32),
            pltpu.SemaphoreType.DMA,
            pltpu.SemaphoreType.DMA,
            pltpu.SemaphoreType.DMA,
            pltpu.SemaphoreType.DMA,
            pltpu.SemaphoreType.DMA,
        ],
    )(gef, nidx_flat, didx_flat)

    logits = pl.pallas_call(
        _mlp_body,
        grid=(B // QB,),
        in_specs=[
            pl.BlockSpec((2 * QB, D), lambda i: (i, 0)),
            pl.BlockSpec((RB, D), lambda i: (i, 0)),
            pl.BlockSpec((2 * D, D), lambda i: (0, 0)),
            pl.BlockSpec((1, D), lambda i: (0, 0)),
            pl.BlockSpec((D, H), lambda i: (0, 0)),
            pl.BlockSpec((1, H), lambda i: (0, 0)),
            pl.BlockSpec((1, H), lambda i: (0, 0)),
            pl.BlockSpec((1, H), lambda i: (0, 0)),
            pl.BlockSpec((H, 1), lambda i: (0, 0)),
        ],
        out_specs=pl.BlockSpec((QB, M), lambda i: (i, 0)),
        out_shape=jax.ShapeDtypeStruct((B, M), f32),
    )(gathered_p, gathered_n, W1, b1.reshape(1, D), Wa,
      ba.reshape(1, H), g.reshape(1, H), beta.reshape(1, H), Wb)

    gum = jax.random.gumbel(jax.random.key(42), (B, M), f32)

    probs, sampled = pl.pallas_call(
        _sample_body,
        out_shape=[jax.ShapeDtypeStruct((B, M), f32),
                   jax.ShapeDtypeStruct((B, 1), jnp.int32)],
    )(logits, bb.reshape(1, 1), gum, neighbor_idx.astype(jnp.int32))

    return (probs, sampled.reshape(B))


# speedup vs baseline: 3.2962x; 1.0276x over previous
import jax
import jax.numpy as jnp
from jax import lax
from jax.experimental import pallas as pl
from jax.experimental.pallas import tpu as pltpu
from jax.experimental.pallas import tpu_sc as plsc

N = 10000
D = 128
B = 1024
M = 64
H = 32
NPAD = 10008

NC, NS = 2, 16
NW = NC * NS
BM = B * M
BP = B * 2
NB_W = BM // NW
NCHUNK = 8
CROWS = NB_W // NCHUNK
PB_W = BP // NW

QB = 64
RB = QB * M


def _sc_gather(t2_hbm, emb_hbm, nidx_hbm, didx_hbm, out_n, out_p,
               idxn_v0, idxn_v1, idxp_v, nbuf0, nbuf1, pair_v,
               semg0, semg1, semw0, semw1, semp):
    wid = lax.axis_index("s") * NC + lax.axis_index("c")
    pltpu.sync_copy(didx_hbm.at[pl.ds(wid * PB_W, PB_W)], idxp_v)
    cpp = pltpu.async_copy(emb_hbm.at[idxp_v], pair_v, semp)
    nbufs = (nbuf0, nbuf1)
    idxvs = (idxn_v0, idxn_v1)
    semgs = (semg0, semg1)
    semws = (semw0, semw1)
    cpg = [None, None]
    cpw = [None, None]
    pltpu.sync_copy(nidx_hbm.at[pl.ds(wid * NB_W, CROWS)], idxvs[0])
    cpg[0] = pltpu.async_copy(t2_hbm.at[idxvs[0]], nbufs[0], semgs[0])
    for c in range(NCHUNK):
        cur = c % 2
        nxt = (c + 1) % 2
        if c + 1 < NCHUNK:
            pltpu.sync_copy(
                nidx_hbm.at[pl.ds(wid * NB_W + (c + 1) * CROWS, CROWS)],
                idxvs[nxt])
            if cpw[nxt] is not None:
                cpw[nxt].wait()
            cpg[nxt] = pltpu.async_copy(
                t2_hbm.at[idxvs[nxt]], nbufs[nxt], semgs[nxt])
        cpg[cur].wait()
        cpw[cur] = pltpu.async_copy(
            nbufs[cur], out_n.at[pl.ds(wid * NB_W + c * CROWS, CROWS)],
            semws[cur])
    cpw[0].wait()
    cpw[1].wait()
    cpp.wait()
    pltpu.sync_copy(pair_v, out_p.at[pl.ds(wid * PB_W, PB_W)])


def _table_body(emb_ref, dw_ref, w1b_ref, t2_ref):
    w1b = w1b_ref[...]
    t2_ref[pl.ds(0, N), :] = jnp.dot(
        emb_ref[...].astype(jnp.bfloat16), w1b,
        preferred_element_type=jnp.float32)
    dwb = jnp.broadcast_to(dw_ref[...], (8, D)).astype(jnp.bfloat16)
    t2_ref[pl.ds(N, 8), :] = jnp.dot(
        dwb, w1b, preferred_element_type=jnp.float32)


def _mlp_body(pair_ref, t2_ref, w1t_ref, b1_ref, wa_ref,
              ba_ref, g_ref, beta_ref, wb_ref, lg_ref):
    pr = pair_ref[...].reshape(QB, 2, D)
    kq = (pr[:, 0, :] + pr[:, 1, :]) * 0.5
    e1b = kq.astype(jnp.bfloat16)
    a1 = jnp.dot(e1b, w1t_ref[...], preferred_element_type=jnp.float32)
    a1r = jnp.broadcast_to(a1[:, None, :], (QB, M, D)).reshape(RB, D)
    kqr = jnp.broadcast_to(kq[:, None, :], (QB, M, D)).reshape(RB, D)
    c1 = a1r + t2_ref[...]
    e = kqr + (c1 + b1_ref[...])
    ebf = e.astype(jnp.bfloat16)
    c2 = jnp.dot(ebf, wa_ref[...], preferred_element_type=jnp.float32)
    h = c2 + ba_ref[...]
    mu = jnp.mean(h, axis=-1, keepdims=True)
    var = jnp.mean((h - mu) ** 2, axis=-1, keepdims=True)
    hn = (h - mu) / jnp.sqrt(var + 1e-5) * g_ref[...] + beta_ref[...]
    t = jnp.tanh(hn)
    c3 = jnp.dot(t, wb_ref[...], preferred_element_type=jnp.float32)
    lg_ref[...] = c3.reshape(QB, M)


def _sample_body(lg_ref, bb_ref, gum_ref, nidx_ref, probs_ref, samp_ref):
    l = lg_ref[...] + bb_ref[0, 0]
    mx = jnp.max(l, axis=-1, keepdims=True)
    ex = jnp.exp(l - mx)
    s = jnp.sum(ex, axis=-1, keepdims=True)
    probs = ex / s
    probs_ref[...] = probs
    z = gum_ref[...] + jnp.log(probs + 1e-20)
    zmax = jnp.max(z, axis=-1, keepdims=True)
    iota = lax.broadcasted_iota(jnp.int32, (B, M), 1)
    samp = jnp.min(jnp.where(z == zmax, iota, M), axis=-1)
    samp_ref[...] = jnp.sum(
        jnp.where(iota == samp[:, None], nidx_ref[...], 0),
        axis=-1, keepdims=True)


def kernel(data_idx, adj_matrix, edge_rel, embeddings, neighbor_idx, done_w,
           W1, b1, Wa, ba, g, beta, Wb, bb):
    del adj_matrix, edge_rel
    f32 = jnp.float32

    t2_tab = pl.pallas_call(
        _table_body,
        out_shape=jax.ShapeDtypeStruct((NPAD, D), f32),
    )(embeddings, done_w, W1[D:, :])

    nidx_flat = neighbor_idx.reshape(BM).astype(jnp.int32)
    didx_flat = data_idx.reshape(BP).astype(jnp.int32)

    mesh = plsc.VectorSubcoreMesh(core_axis_name="c", subcore_axis_name="s")
    gathered_n, gathered_p = pl.kernel(
        _sc_gather,
        mesh=mesh,
        compiler_params=pltpu.CompilerParams(use_tc_tiling_on_sc=False),
        out_type=[jax.ShapeDtypeStruct((BM, D), f32),
                  jax.ShapeDtypeStruct((BP, D), f32)],
        scratch_types=[
            pltpu.VMEM((CROWS,), jnp.int32),
            pltpu.VMEM((CROWS,), jnp.int32),
            pltpu.VMEM((PB_W,), jnp.int32),
            pltpu.VMEM((CROWS, D), f32),
            pltpu.VMEM((CROWS, D), f32),
            pltpu.VMEM((PB_W, D), f32),
            pltpu.SemaphoreType.DMA,
            pltpu.SemaphoreType.DMA,
            pltpu.SemaphoreType.DMA,
            pltpu.SemaphoreType.DMA,
            pltpu.SemaphoreType.DMA,
        ],
    )(t2_tab, embeddings, nidx_flat, didx_flat)

    logits = pl.pallas_call(
        _mlp_body,
        grid=(B // QB,),
        in_specs=[
            pl.BlockSpec((2 * QB, D), lambda i: (i, 0)),
            pl.BlockSpec((RB, D), lambda i: (i, 0)),
            pl.BlockSpec((D, D), lambda i: (0, 0)),
            pl.BlockSpec((1, D), lambda i: (0, 0)),
            pl.BlockSpec((D, H), lambda i: (0, 0)),
            pl.BlockSpec((1, H), lambda i: (0, 0)),
            pl.BlockSpec((1, H), lambda i: (0, 0)),
            pl.BlockSpec((1, H), lambda i: (0, 0)),
            pl.BlockSpec((H, 1), lambda i: (0, 0)),
        ],
        out_specs=pl.BlockSpec((QB, M), lambda i: (i, 0)),
        out_shape=jax.ShapeDtypeStruct((B, M), f32),
    )(gathered_p, gathered_n, W1[:D, :], b1.reshape(1, D), Wa,
      ba.reshape(1, H), g.reshape(1, H), beta.reshape(1, H), Wb)

    gum = jax.random.gumbel(jax.random.key(42), (B, M), f32)

    probs, sampled = pl.pallas_call(
        _sample_body,
        out_shape=[jax.ShapeDtypeStruct((B, M), f32),
                   jax.ShapeDtypeStruct((B, 1), jnp.int32)],
    )(logits, bb.reshape(1, 1), gum, neighbor_idx.astype(jnp.int32))

    return (probs, sampled.reshape(B))


# speedup vs baseline: 3.2964x; 1.0001x over previous
import jax
import jax.numpy as jnp
from jax import lax
from jax.experimental import pallas as pl
from jax.experimental.pallas import tpu as pltpu
from jax.experimental.pallas import tpu_sc as plsc

N = 10000
D = 128
B = 1024
M = 64
H = 32
NPAD = 10008

NC, NS = 2, 16
NW = NC * NS
BM = B * M
BP = B * 2
NB_W = BM // NW
NCHUNK = 8
CROWS = NB_W // NCHUNK
PB_W = BP // NW

QB = 128
RB = QB * M


def _sc_gather(t2_hbm, emb_hbm, nidx_hbm, didx_hbm, out_n, out_p,
               idxn_v0, idxn_v1, idxp_v, nbuf0, nbuf1, pair_v,
               semg0, semg1, semw0, semw1, semp):
    wid = lax.axis_index("s") * NC + lax.axis_index("c")
    pltpu.sync_copy(didx_hbm.at[pl.ds(wid * PB_W, PB_W)], idxp_v)
    cpp = pltpu.async_copy(emb_hbm.at[idxp_v], pair_v, semp)
    nbufs = (nbuf0, nbuf1)
    idxvs = (idxn_v0, idxn_v1)
    semgs = (semg0, semg1)
    semws = (semw0, semw1)
    cpg = [None, None]
    cpw = [None, None]
    pltpu.sync_copy(nidx_hbm.at[pl.ds(wid * NB_W, CROWS)], idxvs[0])
    cpg[0] = pltpu.async_copy(t2_hbm.at[idxvs[0]], nbufs[0], semgs[0])
    for c in range(NCHUNK):
        cur = c % 2
        nxt = (c + 1) % 2
        if c + 1 < NCHUNK:
            pltpu.sync_copy(
                nidx_hbm.at[pl.ds(wid * NB_W + (c + 1) * CROWS, CROWS)],
                idxvs[nxt])
            if cpw[nxt] is not None:
                cpw[nxt].wait()
            cpg[nxt] = pltpu.async_copy(
                t2_hbm.at[idxvs[nxt]], nbufs[nxt], semgs[nxt])
        cpg[cur].wait()
        cpw[cur] = pltpu.async_copy(
            nbufs[cur], out_n.at[pl.ds(wid * NB_W + c * CROWS, CROWS)],
            semws[cur])
    cpw[0].wait()
    cpw[1].wait()
    cpp.wait()
    pltpu.sync_copy(pair_v, out_p.at[pl.ds(wid * PB_W, PB_W)])


def _table_body(emb_ref, dw_ref, w1b_ref, t2_ref):
    w1b = w1b_ref[...]
    t2_ref[pl.ds(0, N), :] = jnp.dot(
        emb_ref[...].astype(jnp.bfloat16), w1b,
        preferred_element_type=jnp.float32)
    dwb = jnp.broadcast_to(dw_ref[...], (8, D)).astype(jnp.bfloat16)
    t2_ref[pl.ds(N, 8), :] = jnp.dot(
        dwb, w1b, preferred_element_type=jnp.float32)


def _mlp_body(pair_ref, t2_ref, w1t_ref, b1_ref, wa_ref,
              ba_ref, g_ref, beta_ref, wb_ref, lg_ref):
    pr = pair_ref[...].reshape(QB, 2, D)
    kq = (pr[:, 0, :] + pr[:, 1, :]) * 0.5
    e1b = kq.astype(jnp.bfloat16)
    a1 = jnp.dot(e1b, w1t_ref[...], preferred_element_type=jnp.float32)
    a1r = jnp.broadcast_to(a1[:, None, :], (QB, M, D)).reshape(RB, D)
    kqr = jnp.broadcast_to(kq[:, None, :], (QB, M, D)).reshape(RB, D)
    c1 = a1r + t2_ref[...]
    e = kqr + (c1 + b1_ref[...])
    ebf = e.astype(jnp.bfloat16)
    c2 = jnp.dot(ebf, wa_ref[...], preferred_element_type=jnp.float32)
    h = c2 + ba_ref[...]
    mu = jnp.mean(h, axis=-1, keepdims=True)
    var = jnp.mean((h - mu) ** 2, axis=-1, keepdims=True)
    hn = (h - mu) / jnp.sqrt(var + 1e-5) * g_ref[...] + beta_ref[...]
    t = jnp.tanh(hn)
    c3 = jnp.dot(t, wb_ref[...], preferred_element_type=jnp.float32)
    lg_ref[...] = c3.reshape(QB, M)


def _sample_body(lg_ref, bb_ref, gum_ref, nidx_ref, probs_ref, samp_ref):
    l = lg_ref[...] + bb_ref[0, 0]
    mx = jnp.max(l, axis=-1, keepdims=True)
    ex = jnp.exp(l - mx)
    s = jnp.sum(ex, axis=-1, keepdims=True)
    probs = ex / s
    probs_ref[...] = probs
    z = gum_ref[...] + jnp.log(probs + 1e-20)
    zmax = jnp.max(z, axis=-1, keepdims=True)
    iota = lax.broadcasted_iota(jnp.int32, (B, M), 1)
    samp = jnp.min(jnp.where(z == zmax, iota, M), axis=-1)
    samp_ref[...] = jnp.sum(
        jnp.where(iota == samp[:, None], nidx_ref[...], 0),
        axis=-1, keepdims=True)


def kernel(data_idx, adj_matrix, edge_rel, embeddings, neighbor_idx, done_w,
           W1, b1, Wa, ba, g, beta, Wb, bb):
    del adj_matrix, edge_rel
    f32 = jnp.float32

    t2_tab = pl.pallas_call(
        _table_body,
        out_shape=jax.ShapeDtypeStruct((NPAD, D), f32),
    )(embeddings, done_w, W1[D:, :])

    nidx_flat = neighbor_idx.reshape(BM).astype(jnp.int32)
    didx_flat = data_idx.reshape(BP).astype(jnp.int32)

    mesh = plsc.VectorSubcoreMesh(core_axis_name="c", subcore_axis_name="s")
    gathered_n, gathered_p = pl.kernel(
        _sc_gather,
        mesh=mesh,
        compiler_params=pltpu.CompilerParams(use_tc_tiling_on_sc=False),
        out_type=[jax.ShapeDtypeStruct((BM, D), f32),
                  jax.ShapeDtypeStruct((BP, D), f32)],
        scratch_types=[
            pltpu.VMEM((CROWS,), jnp.int32),
            pltpu.VMEM((CROWS,), jnp.int32),
            pltpu.VMEM((PB_W,), jnp.int32),
            pltpu.VMEM((CROWS, D), f32),
            pltpu.VMEM((CROWS, D), f32),
            pltpu.VMEM((PB_W, D), f32),
            pltpu.SemaphoreType.DMA,
            pltpu.SemaphoreType.DMA,
            pltpu.SemaphoreType.DMA,
            pltpu.SemaphoreType.DMA,
            pltpu.SemaphoreType.DMA,
        ],
    )(t2_tab, embeddings, nidx_flat, didx_flat)

    logits = pl.pallas_call(
        _mlp_body,
        grid=(B // QB,),
        in_specs=[
            pl.BlockSpec((2 * QB, D), lambda i: (i, 0)),
            pl.BlockSpec((RB, D), lambda i: (i, 0)),
            pl.BlockSpec((D, D), lambda i: (0, 0)),
            pl.BlockSpec((1, D), lambda i: (0, 0)),
            pl.BlockSpec((D, H), lambda i: (0, 0)),
            pl.BlockSpec((1, H), lambda i: (0, 0)),
            pl.BlockSpec((1, H), lambda i: (0, 0)),
            pl.BlockSpec((1, H), lambda i: (0, 0)),
            pl.BlockSpec((H, 1), lambda i: (0, 0)),
        ],
        out_specs=pl.BlockSpec((QB, M), lambda i: (i, 0)),
        out_shape=jax.ShapeDtypeStruct((B, M), f32),
    )(gathered_p, gathered_n, W1[:D, :], b1.reshape(1, D), Wa,
      ba.reshape(1, H), g.reshape(1, H), beta.reshape(1, H), Wb)

    gum = jax.random.gumbel(jax.random.key(42), (B, M), f32)

    probs, sampled = pl.pallas_call(
        _sample_body,
        out_shape=[jax.ShapeDtypeStruct((B, M), f32),
                   jax.ShapeDtypeStruct((B, 1), jnp.int32)],
    )(logits, bb.reshape(1, 1), gum, neighbor_idx.astype(jnp.int32))

    return (probs, sampled.reshape(B))


# speedup vs baseline: 3.3915x; 1.0288x over previous
import jax
import jax.numpy as jnp
from jax import lax
from jax.experimental import pallas as pl
from jax.experimental.pallas import tpu as pltpu
from jax.experimental.pallas import tpu_sc as plsc

N = 10000
D = 128
B = 1024
M = 64
H = 32
NPAD = 10008

NC, NS = 2, 16
NW = NC * NS
BM = B * M
BP = B * 2
NB_W = BM // NW
NCHUNK = 8
CROWS = NB_W // NCHUNK
PB_W = BP // NW

QB = 128
RB = QB * M


def _gather_rows(table, nidx_hbm, out_n, wid, idxvs, nbufs, semgs, semws):
    rows_w = out_n.shape[0] // NW
    nchunk = rows_w // CROWS
    cpg = [None, None]
    cpw = [None, None]
    pltpu.sync_copy(nidx_hbm.at[pl.ds(wid * rows_w, CROWS)], idxvs[0])
    cpg[0] = pltpu.async_copy(table.at[idxvs[0]], nbufs[0], semgs[0])
    for c in range(nchunk):
        cur = c % 2
        nxt = (c + 1) % 2
        if c + 1 < nchunk:
            pltpu.sync_copy(
                nidx_hbm.at[pl.ds(wid * rows_w + (c + 1) * CROWS, CROWS)],
                idxvs[nxt])
            if cpw[nxt] is not None:
                cpw[nxt].wait()
            cpg[nxt] = pltpu.async_copy(
                table.at[idxvs[nxt]], nbufs[nxt], semgs[nxt])
        cpg[cur].wait()
        cpw[cur] = pltpu.async_copy(
            nbufs[cur], out_n.at[pl.ds(wid * rows_w + c * CROWS, CROWS)],
            semws[cur])
    cpw[0].wait()
    cpw[1].wait()


def _sc_gather_a(t2_hbm, emb_hbm, nidx_hbm, didx_hbm, out_n, out_p,
                 idxn_v0, idxn_v1, idxp_v, nbuf0, nbuf1, pair_v,
                 semg0, semg1, semw0, semw1, semp):
    wid = lax.axis_index("s") * NC + lax.axis_index("c")
    pltpu.sync_copy(didx_hbm.at[pl.ds(wid * PB_W, PB_W)], idxp_v)
    cpp = pltpu.async_copy(emb_hbm.at[idxp_v], pair_v, semp)
    _gather_rows(t2_hbm, nidx_hbm, out_n, wid, (idxn_v0, idxn_v1),
                 (nbuf0, nbuf1), (semg0, semg1), (semw0, semw1))
    cpp.wait()
    pltpu.sync_copy(pair_v, out_p.at[pl.ds(wid * PB_W, PB_W)])


def _sc_gather_b(t2_hbm, nidx_hbm, out_n,
                 idxn_v0, idxn_v1, nbuf0, nbuf1,
                 semg0, semg1, semw0, semw1):
    wid = lax.axis_index("s") * NC + lax.axis_index("c")
    _gather_rows(t2_hbm, nidx_hbm, out_n, wid, (idxn_v0, idxn_v1),
                 (nbuf0, nbuf1), (semg0, semg1), (semw0, semw1))


def _table_body(emb_ref, dw_ref, w1b_ref, t2_ref):
    w1b = w1b_ref[...]
    t2_ref[pl.ds(0, N), :] = jnp.dot(
        emb_ref[...].astype(jnp.bfloat16), w1b,
        preferred_element_type=jnp.float32)
    dwb = jnp.broadcast_to(dw_ref[...], (8, D)).astype(jnp.bfloat16)
    t2_ref[pl.ds(N, 8), :] = jnp.dot(
        dwb, w1b, preferred_element_type=jnp.float32)


def _mlp_body(pair_ref, t2_ref, w1t_ref, b1_ref, wa_ref,
              ba_ref, g_ref, beta_ref, wb_ref, lg_ref):
    pr = pair_ref[...].reshape(QB, 2, D)
    kq = (pr[:, 0, :] + pr[:, 1, :]) * 0.5
    e1b = kq.astype(jnp.bfloat16)
    a1 = jnp.dot(e1b, w1t_ref[...], preferred_element_type=jnp.float32)
    a1r = jnp.broadcast_to(a1[:, None, :], (QB, M, D)).reshape(RB, D)
    kqr = jnp.broadcast_to(kq[:, None, :], (QB, M, D)).reshape(RB, D)
    c1 = a1r + t2_ref[...]
    e = kqr + (c1 + b1_ref[...])
    ebf = e.astype(jnp.bfloat16)
    c2 = jnp.dot(ebf, wa_ref[...], preferred_element_type=jnp.float32)
    h = c2 + ba_ref[...]
    mu = jnp.mean(h, axis=-1, keepdims=True)
    var = jnp.mean((h - mu) ** 2, axis=-1, keepdims=True)
    hn = (h - mu) / jnp.sqrt(var + 1e-5) * g_ref[...] + beta_ref[...]
    t = jnp.tanh(hn)
    c3 = jnp.dot(t, wb_ref[...], preferred_element_type=jnp.float32)
    lg_ref[...] = c3.reshape(QB, M)


def _sample_body(lg_ref, bb_ref, gum_ref, nidx_ref, probs_ref, samp_ref):
    l = lg_ref[...] + bb_ref[0, 0]
    mx = jnp.max(l, axis=-1, keepdims=True)
    ex = jnp.exp(l - mx)
    s = jnp.sum(ex, axis=-1, keepdims=True)
    probs = ex / s
    probs_ref[...] = probs
    z = gum_ref[...] + jnp.log(probs + 1e-20)
    zmax = jnp.max(z, axis=-1, keepdims=True)
    iota = lax.broadcasted_iota(jnp.int32, (B, M), 1)
    samp = jnp.min(jnp.where(z == zmax, iota, M), axis=-1)
    samp_ref[...] = jnp.sum(
        jnp.where(iota == samp[:, None], nidx_ref[...], 0),
        axis=-1, keepdims=True)


def kernel(data_idx, adj_matrix, edge_rel, embeddings, neighbor_idx, done_w,
           W1, b1, Wa, ba, g, beta, Wb, bb):
    del adj_matrix, edge_rel
    f32 = jnp.float32

    t2_tab = pl.pallas_call(
        _table_body,
        out_shape=jax.ShapeDtypeStruct((NPAD, D), f32),
    )(embeddings, done_w, W1[D:, :])

    nidx_flat = neighbor_idx.reshape(BM).astype(jnp.int32)
    didx_flat = data_idx.reshape(BP).astype(jnp.int32)
    BH = BM // 2

    mesh = plsc.VectorSubcoreMesh(core_axis_name="c", subcore_axis_name="s")
    sc_params = pltpu.CompilerParams(use_tc_tiling_on_sc=False)
    nbuf_scratch = [
        pltpu.VMEM((CROWS,), jnp.int32),
        pltpu.VMEM((CROWS,), jnp.int32),
        pltpu.VMEM((CROWS, D), f32),
        pltpu.VMEM((CROWS, D), f32),
        pltpu.SemaphoreType.DMA,
        pltpu.SemaphoreType.DMA,
        pltpu.SemaphoreType.DMA,
        pltpu.SemaphoreType.DMA,
    ]
    gathered_a, gathered_p = pl.kernel(
        _sc_gather_a,
        mesh=mesh,
        compiler_params=sc_params,
        out_type=[jax.ShapeDtypeStruct((BH, D), f32),
                  jax.ShapeDtypeStruct((BP, D), f32)],
        scratch_types=nbuf_scratch[:2] + [pltpu.VMEM((PB_W,), jnp.int32)]
        + nbuf_scratch[2:4] + [pltpu.VMEM((PB_W, D), f32)]
        + nbuf_scratch[4:] + [pltpu.SemaphoreType.DMA],
    )(t2_tab, embeddings, nidx_flat[:BH], didx_flat)

    gathered_b = pl.kernel(
        _sc_gather_b,
        mesh=mesh,
        compiler_params=sc_params,
        out_type=jax.ShapeDtypeStruct((BH, D), f32),
        scratch_types=nbuf_scratch,
    )(t2_tab, nidx_flat[BH:])

    nq = B // 2 // QB
    mlp_specs = dict(
        grid=(nq,),
        in_specs=[
            pl.BlockSpec((2 * QB, D), lambda i: (i, 0)),
            pl.BlockSpec((RB, D), lambda i: (i, 0)),
            pl.BlockSpec((D, D), lambda i: (0, 0)),
            pl.BlockSpec((1, D), lambda i: (0, 0)),
            pl.BlockSpec((D, H), lambda i: (0, 0)),
            pl.BlockSpec((1, H), lambda i: (0, 0)),
            pl.BlockSpec((1, H), lambda i: (0, 0)),
            pl.BlockSpec((1, H), lambda i: (0, 0)),
            pl.BlockSpec((H, 1), lambda i: (0, 0)),
        ],
        out_specs=pl.BlockSpec((QB, M), lambda i: (i, 0)),
        out_shape=jax.ShapeDtypeStruct((B // 2, M), f32),
    )
    wargs = (W1[:D, :], b1.reshape(1, D), Wa, ba.reshape(1, H),
             g.reshape(1, H), beta.reshape(1, H), Wb)
    logits_a = pl.pallas_call(_mlp_body, **mlp_specs)(
        gathered_p[:BP // 2], gathered_a, *wargs)
    logits_b = pl.pallas_call(_mlp_body, **mlp_specs)(
        gathered_p[BP // 2:], gathered_b, *wargs)
    logits = jnp.concatenate([logits_a, logits_b], axis=0)

    gum = jax.random.gumbel(jax.random.key(42), (B, M), f32)

    probs, sampled = pl.pallas_call(
        _sample_body,
        out_shape=[jax.ShapeDtypeStruct((B, M), f32),
                   jax.ShapeDtypeStruct((B, 1), jnp.int32)],
    )(logits, bb.reshape(1, 1), gum, neighbor_idx.astype(jnp.int32))

    return (probs, sampled.reshape(B))
